# Initial kernel scaffold; baseline (speedup 1.0000x reference)
import jax
import jax.numpy as jnp
from jax.experimental import pallas as pl


def kernel(feats, coords, W):
    raise NotImplementedError("write your pallas kernel here")


# speedup vs baseline: 3.6037x; 3.6037x over previous
import jax
import jax.numpy as jnp
from jax.experimental import pallas as pl
from jax.experimental.pallas import tpu as pltpu

_G = 128
_CIN = 2
_COUT = 16


def _conv_body(w_ref, pa_ref, pb_ref, pc_ref, out_ref):
    planes = []
    for ref in (pa_ref, pb_ref, pc_ref):
        for c in range(_CIN):
            planes.append(ref[c, 0])

    subs = {}
    for dx in range(3):
        for c in range(_CIN):
            p = planes[dx * _CIN + c]
            for dy in range(3):
                for dz in range(3):
                    subs[(dx, dy, dz, c)] = jax.lax.slice(
                        p, (dy, dz), (dy + _G, dz + _G))

    for o in range(_COUT):
        acc = jnp.zeros((_G, _G), jnp.float32)
        for dx in range(3):
            for dy in range(3):
                for dz in range(3):
                    k = dx * 9 + dy * 3 + dz
                    for c in range(_CIN):
                        acc = acc + subs[(dx, dy, dz, c)] * w_ref[k * (_CIN * _COUT) + c * _COUT + o]
        out_ref[0, o] = acc


def kernel(feats, coords, W):
    n = feats.shape[0]

    keys = (coords[:, 0] * _G + coords[:, 1]) * _G + coords[:, 2]
    rep = jnp.full((_G * _G * _G,), n, dtype=jnp.int32)
    rep = rep.at[keys].min(jnp.arange(n, dtype=jnp.int32))
    occupied = rep < n
    grid_feats = jnp.where(occupied[:, None],
                           feats[jnp.clip(rep, 0, n - 1)], 0.0)
    grid_feats = grid_feats.reshape(_G, _G, _G, _CIN).transpose(3, 0, 1, 2)
    padded = jnp.pad(grid_feats, ((0, 0), (1, 1), (1, 1), (1, 1)))

    w_flat = W.astype(jnp.float32).reshape(-1)

    conv = pl.pallas_call(
        _conv_body,
        grid=(_G,),
        in_specs=[
            pl.BlockSpec(memory_space=pltpu.SMEM),
            pl.BlockSpec((_CIN, 1, _G + 2, _G + 2), lambda i: (0, i, 0, 0)),
            pl.BlockSpec((_CIN, 1, _G + 2, _G + 2), lambda i: (0, i + 1, 0, 0)),
            pl.BlockSpec((_CIN, 1, _G + 2, _G + 2), lambda i: (0, i + 2, 0, 0)),
        ],
        out_specs=pl.BlockSpec((1, _COUT, _G, _G), lambda i: (i, 0, 0, 0)),
        out_shape=jax.ShapeDtypeStruct((_G, _COUT, _G, _G), jnp.float32),
    )(w_flat, padded, padded, padded)

    return conv[coords[:, 0], :, coords[:, 1], coords[:, 2]]
